# Initial kernel scaffold; baseline (speedup 1.0000x reference)
import jax
import jax.numpy as jnp
from jax.experimental import pallas as pl


def kernel(pcd):
    raise NotImplementedError("write your pallas kernel here")


# speedup vs baseline: 43.7591x; 43.7591x over previous
import jax
import jax.numpy as jnp
from jax.experimental import pallas as pl
from jax.experimental.pallas import tpu as pltpu

_B = 16
_N = 2048
_H = 0.0005
_ROWS = 256


def _tc_body(xr_ref, xc_ref, out_ref):
    b = pl.program_id(0)
    r = pl.program_id(1)

    xr = xr_ref[0]
    xc = xc_ref[0]
    sq_r = jnp.sum(xr * xr, axis=1, keepdims=True)
    sq_c = jnp.sum(xc * xc, axis=0, keepdims=True)
    g = jax.lax.dot_general(
        xr.astype(jnp.bfloat16), xc.astype(jnp.bfloat16), (((1,), (0,)), ((), ())),
        preferred_element_type=jnp.float32,
    )
    d2 = jnp.maximum(sq_r + sq_c - 2.0 * g, 0.0)

    f = jnp.maximum(_H - d2 * d2, 0.0)
    iszero = d2 == 0.0
    n0 = jnp.sum(iszero.astype(jnp.float32), axis=1, keepdims=True)
    fp = jnp.where(iszero, 0.0, f)
    m1 = jnp.max(fp, axis=1, keepdims=True)
    f1 = jnp.where(fp == m1, 0.0, fp)
    m2 = jnp.max(f1, axis=1, keepdims=True)
    f2 = jnp.where(f1 == m2, 0.0, f1)
    m3 = jnp.max(f2, axis=1, keepdims=True)
    f3 = jnp.where(f2 == m3, 0.0, f2)
    m4 = jnp.max(f3, axis=1, keepdims=True)
    f4 = jnp.where(f3 == m4, 0.0, f3)
    m5 = jnp.max(f4, axis=1, keepdims=True)

    nz5 = jnp.minimum(n0, 5.0)
    npos = 5.0 - nz5
    s_pos = (jnp.where(npos >= 1, m1, 0.0) + jnp.where(npos >= 2, m2, 0.0)
             + jnp.where(npos >= 3, m3, 0.0) + jnp.where(npos >= 4, m4, 0.0)
             + jnp.where(npos >= 5, m5, 0.0))
    maxf = jnp.where(n0 > 0, _H, m1)
    contrib = jnp.sum(nz5 * _H + s_pos - maxf)

    @pl.when(jnp.logical_and(b == 0, r == 0))
    def _():
        out_ref[0, 0] = 0.0

    out_ref[0, 0] += contrib


def kernel(pcd):
    xt = jnp.transpose(pcd, (0, 2, 1))
    total = pl.pallas_call(
        _tc_body,
        grid=(_B, _N // _ROWS),
        in_specs=[
            pl.BlockSpec((1, _ROWS, 3), lambda b, r: (b, r, 0)),
            pl.BlockSpec((1, 3, _N), lambda b, r: (b, 0, 0)),
        ],
        out_specs=pl.BlockSpec(memory_space=pltpu.SMEM),
        out_shape=jax.ShapeDtypeStruct((1, 1), jnp.float32),
    )(xt, pcd)
    return total[0, 0] / (_B * _N * 4)
